# baseline (speedup 1.0000x reference)
import functools

import jax
import jax.numpy as jnp
from jax import lax
from jax.experimental import pallas as pl
from jax.experimental.pallas import tpu as pltpu
from jax.experimental.pallas import tpu_sc as plsc


def _make_sc_broadcast(S0, S, N, D, dtype):
    rows = S - S0
    info = plsc.get_sparse_core_info()
    num_workers = info.num_cores * info.num_subcores
    rows_per_w = rows // num_workers
    chunk = min(32, rows_per_w)
    n_chunks = rows_per_w // chunk
    mesh = plsc.VectorSubcoreMesh(core_axis_name="c", subcore_axis_name="s")

    @functools.partial(
        pl.kernel,
        mesh=mesh,
        out_type=jax.ShapeDtypeStruct((rows, N, D), dtype),
        scratch_types=[
            pltpu.VMEM((chunk, D), dtype),
            pltpu.VMEM((chunk, D), dtype),
            pltpu.SemaphoreType.DMA,
            pltpu.SemaphoreType.DMA,
            pltpu.SemaphoreType.DMA,
            pltpu.SemaphoreType.DMA,
        ],
    )
    def sc_kernel(table_hbm, out_hbm, buf0, buf1, rsem0, rsem1, wsem0, wsem1):
        wid = lax.axis_index("s") * info.num_cores + lax.axis_index("c")
        base = wid * rows_per_w
        bufs, rsems, wsems = [buf0, buf1], [rsem0, rsem1], [wsem0, wsem1]

        def src(i):
            return table_hbm.at[pl.ds(S0 + base + i * chunk, chunk)]

        reads = {
            0: pltpu.async_copy(src(0), buf0, rsem0),
        }
        if n_chunks > 1:
            reads[1] = pltpu.async_copy(src(1), buf1, rsem1)
        tail_writes = []
        for i in range(n_chunks):
            b = i % 2
            reads[i].wait()
            writes = [
                pltpu.async_copy(
                    bufs[b], out_hbm.at[pl.ds(base + i * chunk, chunk), n], wsems[b]
                )
                for n in range(N)
            ]
            if i + 2 < n_chunks:
                for h in writes:
                    h.wait()
                reads[i + 2] = pltpu.async_copy(src(i + 2), bufs[b], rsems[b])
            else:
                tail_writes.extend(writes)
        for h in tail_writes:
            h.wait()

    return sc_kernel


def _tc_copy(S0, S, N, D, dtype, bs=256):
    rows = S - S0

    def body(tab_ref, out_ref):
        r = tab_ref[...]
        out_ref[...] = jnp.broadcast_to(r[:, None, :], (bs, N, D))

    return pl.pallas_call(
        body,
        grid=(rows // bs,),
        in_specs=[pl.BlockSpec((bs, D), lambda i: (i + S0 // bs, 0))],
        out_specs=pl.BlockSpec((bs, N, D), lambda i: (i, 0, 0)),
        out_shape=jax.ShapeDtypeStruct((rows, N, D), dtype),
    )


def kernel(x, pos_embedding):
    S, N = x.shape
    _, D = pos_embedding.shape
    K = 2048
    sc_out = _make_sc_broadcast(0, K, N, D, pos_embedding.dtype)(pos_embedding)
    tc_out = _tc_copy(K, S, N, D, pos_embedding.dtype)(pos_embedding)
    return jnp.concatenate([sc_out, tc_out], axis=0)


# speedup vs baseline: 2.8339x; 2.8339x over previous
import functools

import jax
import jax.numpy as jnp
from jax import lax
from jax.experimental import pallas as pl
from jax.experimental.pallas import tpu as pltpu
from jax.experimental.pallas import tpu_sc as plsc


def _make_sc_broadcast(S, N, D, dtype):
    info = plsc.get_sparse_core_info()
    num_workers = info.num_cores * info.num_subcores
    rows_per_w = S // num_workers
    chunk = min(32, rows_per_w)
    n_chunks = rows_per_w // chunk
    mesh = plsc.VectorSubcoreMesh(core_axis_name="c", subcore_axis_name="s")

    @functools.partial(
        pl.kernel,
        mesh=mesh,
        out_type=jax.ShapeDtypeStruct((S, N, D), dtype),
        scratch_types=[
            pltpu.VMEM((chunk, D), dtype),
            pltpu.VMEM((chunk, D), dtype),
            pltpu.SemaphoreType.DMA,
            pltpu.SemaphoreType.DMA,
            pltpu.SemaphoreType.DMA,
            pltpu.SemaphoreType.DMA,
        ],
    )
    def sc_kernel(table_hbm, out_hbm, buf0, buf1, rsem0, rsem1, wsem0, wsem1):
        wid = lax.axis_index("s") * info.num_cores + lax.axis_index("c")
        base = wid * rows_per_w
        bufs, rsems, wsems = [buf0, buf1], [rsem0, rsem1], [wsem0, wsem1]

        def src(i):
            return table_hbm.at[pl.ds(base + i * chunk, chunk)]

        reads = {
            0: pltpu.async_copy(src(0), buf0, rsem0),
            1: pltpu.async_copy(src(1), buf1, rsem1),
        }
        tail_writes = []
        for i in range(n_chunks):
            b = i % 2
            reads[i].wait()
            writes = [
                pltpu.async_copy(
                    bufs[b], out_hbm.at[pl.ds(base + i * chunk, chunk), n], wsems[b]
                )
                for n in range(N)
            ]
            if i + 2 < n_chunks:
                for h in writes:
                    h.wait()
                reads[i + 2] = pltpu.async_copy(src(i + 2), bufs[b], rsems[b])
            else:
                tail_writes.extend(writes)
        for h in tail_writes:
            h.wait()

    return sc_kernel


def kernel(x, pos_embedding):
    S, N = x.shape
    _, D = pos_embedding.shape
    return _make_sc_broadcast(S, N, D, pos_embedding.dtype)(pos_embedding)
